# baseline (speedup 1.0000x reference)
import jax
import jax.numpy as jnp
from jax.experimental import pallas as pl
from jax.experimental.pallas import tpu as pltpu

_BSZ = 128
_NSEM = 16


def _body(t_ref, o_ref, sems):
    copies = [
        pltpu.make_async_copy(t_ref, o_ref.at[i], sems.at[i % _NSEM])
        for i in range(_BSZ)
    ]
    for c in copies:
        c.start()
    for c in copies:
        c.wait()


def kernel(batch_size, table):
    n, d = table.shape
    return pl.pallas_call(
        _body,
        in_specs=[pl.BlockSpec(memory_space=pltpu.HBM)],
        out_specs=pl.BlockSpec(memory_space=pltpu.HBM),
        out_shape=jax.ShapeDtypeStruct((_BSZ, n, d), table.dtype),
        scratch_shapes=[
            pltpu.SemaphoreType.DMA((_NSEM,)),
        ],
    )(table)


# speedup vs baseline: 29.8952x; 29.8952x over previous
import functools
import jax
import jax.numpy as jnp
from jax import lax
from jax.experimental import pallas as pl
from jax.experimental.pallas import tpu as pltpu
from jax.experimental.pallas import tpu_sc as plsc

_B = 128
_NC = 2
_NS = 16
_NW = _NC * _NS


def _sc_body(n_per_w, table_hbm, out_hbm, slice_v, sem):
    wid = lax.axis_index("s") * _NC + lax.axis_index("c")
    base = wid * n_per_w
    pltpu.sync_copy(table_hbm.at[pl.ds(base, n_per_w)], slice_v)
    copies = [
        pltpu.make_async_copy(
            slice_v, out_hbm.at[b, pl.ds(base, n_per_w)], sem
        )
        for b in range(_B)
    ]
    for c in copies:
        c.start()
    for c in copies:
        c.wait()


def kernel(batch_size, table):
    n, d = table.shape
    n_per_w = n // _NW
    mesh = plsc.VectorSubcoreMesh(core_axis_name="c", subcore_axis_name="s")
    k = pl.kernel(
        functools.partial(_sc_body, n_per_w),
        out_type=jax.ShapeDtypeStruct((_B, n, d), table.dtype),
        mesh=mesh,
        scratch_types=[
            pltpu.VMEM((n_per_w, d), table.dtype),
            pltpu.SemaphoreType.DMA,
        ],
        compiler_params=pltpu.CompilerParams(use_tc_tiling_on_sc=True),
    )
    return k(table)
